# Initial kernel scaffold; baseline (speedup 1.0000x reference)
import jax
import jax.numpy as jnp
from jax.experimental import pallas as pl


def kernel(h, m_rbf, idnb_a, idnb_c, W):
    raise NotImplementedError("write your pallas kernel here")


# speedup vs baseline: 2.3861x; 2.3861x over previous
import functools

import jax
import jax.numpy as jnp
from jax import lax
from jax.experimental import pallas as pl
from jax.experimental.pallas import tpu as pltpu
from jax.experimental.pallas import tpu_sc as plsc

N_NODES = 10000
N_EDGES = 320000
D_ATOM = 128
D_EDGE = 16
D_OUT = 128

NC = 2
NS = 16
NW = NC * NS
E_PER_W = N_EDGES // NW
B = 80
STEPS = E_PER_W // B


def _node_mm_body(h_ref, wa_ref, wc_ref, pa_ref, pc_ref):
    hb = h_ref[...]
    pa_ref[...] = jnp.dot(hb, wa_ref[...], preferred_element_type=jnp.float32)
    pc_ref[...] = jnp.dot(hb, wc_ref[...], preferred_element_type=jnp.float32)


def _node_mm(h, wa, wc):
    blk = 2000
    return pl.pallas_call(
        _node_mm_body,
        grid=(N_NODES // blk,),
        in_specs=[
            pl.BlockSpec((blk, D_ATOM), lambda i: (i, 0)),
            pl.BlockSpec((D_ATOM, D_OUT), lambda i: (0, 0)),
            pl.BlockSpec((D_ATOM, D_OUT), lambda i: (0, 0)),
        ],
        out_specs=[
            pl.BlockSpec((blk, D_OUT), lambda i: (i, 0)),
            pl.BlockSpec((blk, D_OUT), lambda i: (i, 0)),
        ],
        out_shape=[
            jax.ShapeDtypeStruct((N_NODES, D_OUT), jnp.float32),
            jax.ShapeDtypeStruct((N_NODES, D_OUT), jnp.float32),
        ],
    )(h, wa, wc)


def _rbf_mm_body(m_ref, wr_ref, r_ref):
    r_ref[...] = jnp.dot(m_ref[...], wr_ref[...], preferred_element_type=jnp.float32)


def _rbf_mm(m_rbf, wr):
    blk = 4000
    return pl.pallas_call(
        _rbf_mm_body,
        grid=(N_EDGES // blk,),
        in_specs=[
            pl.BlockSpec((blk, D_EDGE), lambda i: (i, 0)),
            pl.BlockSpec((D_EDGE, D_OUT), lambda i: (0, 0)),
        ],
        out_specs=pl.BlockSpec((blk, D_OUT), lambda i: (i, 0)),
        out_shape=jax.ShapeDtypeStruct((N_EDGES, D_OUT), jnp.float32),
    )(m_rbf, wr)


_MESH = plsc.VectorSubcoreMesh(core_axis_name="c", subcore_axis_name="s")


@functools.partial(
    pl.kernel,
    mesh=_MESH,
    out_type=jax.ShapeDtypeStruct((N_EDGES, D_OUT), jnp.float32),
    scratch_types=[
        pltpu.VMEM((B,), jnp.int32),
        pltpu.VMEM((B,), jnp.int32),
        pltpu.VMEM((B, D_OUT), jnp.float32),
        pltpu.VMEM((B, D_OUT), jnp.float32),
        pltpu.VMEM((B, D_OUT), jnp.float32),
        pltpu.VMEM((B, D_OUT), jnp.float32),
        pltpu.SemaphoreType.DMA,
        pltpu.SemaphoreType.DMA,
        pltpu.SemaphoreType.DMA,
    ],
)
def _edge_kernel(pa_hbm, pc_hbm, r_hbm, ia_hbm, ic_hbm, out_hbm,
                 idxa_v, idxc_v, accr_v, bufa_v, bufc_v, outb_v,
                 sem_r, sem_a, sem_c):
    wid = lax.axis_index("s") * NC + lax.axis_index("c")
    base0 = wid * E_PER_W

    def step(i, carry):
        base = base0 + i * B
        pltpu.sync_copy(ia_hbm.at[pl.ds(base, B)], idxa_v)
        pltpu.sync_copy(ic_hbm.at[pl.ds(base, B)], idxc_v)
        cp_a = pltpu.async_copy(pa_hbm.at[idxa_v], bufa_v, sem_a)
        cp_c = pltpu.async_copy(pc_hbm.at[idxc_v], bufc_v, sem_c)
        cp_r = pltpu.async_copy(r_hbm.at[pl.ds(base, B)], accr_v, sem_r)
        cp_r.wait()
        cp_a.wait()
        cp_c.wait()

        def row(j, c2):
            for k in range(D_OUT // 16):
                sl = pl.ds(k * 16, 16)
                x = accr_v[j, sl] + bufa_v[j, sl] + bufc_v[j, sl]
                outb_v[j, sl] = x / (1.0 + jnp.exp(-x))
            return c2

        lax.fori_loop(0, B, row, 0)
        pltpu.sync_copy(outb_v, out_hbm.at[pl.ds(base, B)])
        return carry

    lax.fori_loop(0, STEPS, step, 0)


def kernel(h, m_rbf, idnb_a, idnb_c, W):
    wa = W[:D_ATOM]
    wc = W[D_ATOM:2 * D_ATOM]
    wr = W[2 * D_ATOM:]
    pa, pc = _node_mm(h, wa, wc)
    r = _rbf_mm(m_rbf, wr)
    return _edge_kernel(pa, pc, r, idnb_a, idnb_c)
